# baseline (speedup 1.0000x reference)
import functools

import jax
import jax.numpy as jnp
from jax import lax
from jax.experimental import pallas as pl
from jax.experimental.pallas import tpu as pltpu
from jax.experimental.pallas import tpu_sc as plsc

_ROWS = 16384
_COLS = 50
_D = 64
_NC, _NS = 2, 16
_NW = _NC * _NS
_RPW = _ROWS // _NW
_NBUF = 8

_mesh = plsc.VectorSubcoreMesh(core_axis_name="c", subcore_axis_name="s")


@functools.partial(
    pl.kernel,
    out_type=jax.ShapeDtypeStruct((_ROWS, _COLS, _D), jnp.float32),
    mesh=_mesh,
    scratch_types=[
        pltpu.VMEM((_RPW, _COLS), jnp.int32),
        pltpu.VMEM((_NBUF, _COLS, _D), jnp.float32),
        [pltpu.SemaphoreType.DMA] * _NBUF,
        [pltpu.SemaphoreType.DMA] * _NBUF,
    ],
    compiler_params=pltpu.CompilerParams(use_tc_tiling_on_sc=False),
)
def _emb_gather(idx_hbm, table_hbm, out_hbm, idx_v, rows_v, gsems, fsems):
    wid = lax.axis_index("s") * _NC + lax.axis_index("c")
    base = wid * _RPW

    pltpu.sync_copy(idx_hbm.at[pl.ds(base, _RPW)], idx_v)

    def gather_copy(j, b):
        return pltpu.make_async_copy(
            table_hbm.at[idx_v.at[j]], rows_v.at[b], gsems[b]
        )

    def flush_copy(j, b):
        return pltpu.make_async_copy(
            rows_v.at[b], out_hbm.at[base + j], fsems[b]
        )

    for b in range(_NBUF):
        gather_copy(b, b).start()

    @pl.loop(0, _RPW, step=_NBUF)
    def _(j):
        for b in range(_NBUF):
            gather_copy(j + b, b).wait()
            flush_copy(j + b, b).start()
        for b in range(_NBUF):
            flush_copy(j + b, b).wait()

            @pl.when(j + _NBUF + b < _RPW)
            def _():
                gather_copy(j + _NBUF + b, b).start()


def kernel(x, table):
    return _emb_gather(x, table)
